# Initial kernel scaffold; baseline (speedup 1.0000x reference)
import jax
import jax.numpy as jnp
from jax.experimental import pallas as pl


def kernel(xs, scale, mip0, mip1, mip2, mip3, mip4, mip5, mip6, mip7, mip8, mip9, mip10):
    raise NotImplementedError("write your pallas kernel here")


# speedup vs baseline: 37.0394x; 37.0394x over previous
import functools
import math

import jax
import jax.numpy as jnp
from jax import lax
from jax.experimental import pallas as pl
from jax.experimental.pallas import tpu as pltpu
from jax.experimental.pallas import tpu_sc as plsc

N = 1048576
LEVELS = 11
T = sum((1024 >> i) ** 2 for i in range(LEVELS))

NC, NS = 2, 16
NW = NC * NS
PTS_PER_W = N // NW
C = 1024
G = PTS_PER_W // C
SUB = 128
NSUB = C // SUB

_LN2 = math.log(2.0)
_C1, _C3, _C5, _C7 = 2 / _LN2, 2 / (3 * _LN2), 2 / (5 * _LN2), 2 / (7 * _LN2)


def _body(scale_hbm, x_hbm, y_hbm, w_hbm, out_hbm,
          sc_v, xv, yv,
          idx_a0, idx_a1, idx_b0, idx_b1,
          rb_a0, rb_a1, rb_b0, rb_b1,
          wxa, wya, wxb, wyb, dxa, dxb,
          outb, sem):
    wid = lax.axis_index("s") * NC + lax.axis_index("c")
    base = wid * PTS_PER_W

    pltpu.sync_copy(scale_hbm, sc_v)
    s = sc_v[...]
    bits = lax.bitcast_convert_type(s, jnp.int32)
    e = (bits >> 23) - 127
    mant = lax.bitcast_convert_type(
        (bits & 0x007FFFFF) | 0x3F800000, jnp.float32)
    z = (mant - 1.0) / (mant + 1.0)
    z2 = z * z
    log2m = z * (_C1 + z2 * (_C3 + z2 * (_C5 + z2 * _C7)))
    mi = jnp.clip(e.astype(jnp.float32) + log2m, 0.0, 10.0)
    m0 = mi.astype(jnp.int32)
    wl = mi - m0.astype(jnp.float32)
    m1 = jnp.minimum(m0 + 1, 10)

    def lvl_params(m):
        wi = jnp.right_shift(jnp.full((16,), 1024, jnp.int32), m)
        off = (jnp.full((16,), 0x55555555, jnp.int32)
               & ((jnp.full((16,), 1, jnp.int32) << (2 * m)) - 1)) << (22 - 2 * m)
        return wi, wi.astype(jnp.float32), off

    wia, wfa, offa = lvl_params(m0)
    wib, wfb, offb = lvl_params(m1)

    levels = ((wia, wfa, offa, idx_a0, idx_a1, wxa, wya, dxa),
              (wib, wfb, offb, idx_b0, idx_b1, wxb, wyb, dxb))

    def chunk(g, _):
        cb = base + g * C
        pltpu.sync_copy(x_hbm.at[pl.ds(cb, C)], xv)
        pltpu.sync_copy(y_hbm.at[pl.ds(cb, C)], yv)

        def index_pass(j, _):
            xg = xv[pl.ds(j * 16, 16)]
            yg = yv[pl.ds(j * 16, 16)]
            for (wi, wf, off, iA, iB, wxv, wyv, dxv) in levels:
                mx = xg * wf
                ix = mx.astype(jnp.int32)
                fx = mx - ix.astype(jnp.float32)
                x0 = jnp.maximum(jnp.minimum(ix, wi - 1), 0)
                x1 = jnp.minimum(x0 + 1, wi - 1)
                my = yg * wf
                iy = my.astype(jnp.int32)
                fy = my - iy.astype(jnp.float32)
                y0 = jnp.maximum(jnp.minimum(iy, wi - 1), 0)
                y1 = jnp.minimum(y0 + 1, wi - 1)
                iA[pl.ds(j * 16, 16)] = off + y0 * wi + x0
                iB[pl.ds(j * 16, 16)] = off + y1 * wi + x0
                wxv[pl.ds(j * 16, 16)] = fx
                wyv[pl.ds(j * 16, 16)] = fy
                dxv[pl.ds(j * 16, 16)] = x1 - x0
            return 0

        lax.fori_loop(0, C // 16, index_pass, 0)

        copies = []
        for (ibuf, rbuf) in ((idx_a0, rb_a0), (idx_a1, rb_a1),
                             (idx_b0, rb_b0), (idx_b1, rb_b1)):
            for k in range(NSUB):
                copies.append(pltpu.async_copy(
                    w_hbm.at[ibuf.at[pl.ds(k * SUB, SUB)]],
                    rbuf.at[pl.ds(k * SUB, SUB)], sem))
        for cp in copies:
            cp.wait()

        def combine(j, _):
            rows = j * 16 + lax.iota(jnp.int32, 16)
            fxa = wxa[pl.ds(j * 16, 16)]
            fya = wya[pl.ds(j * 16, 16)]
            fxb = wxb[pl.ds(j * 16, 16)]
            fyb = wyb[pl.ds(j * 16, 16)]
            ca = dxa[pl.ds(j * 16, 16)] * 4
            cb2 = dxb[pl.ds(j * 16, 16)] * 4
            for c in range(3):
                cc = jnp.full((16,), c, jnp.int32)
                v00 = plsc.load_gather(rb_a0, [rows, cc])
                v01 = plsc.load_gather(rb_a0, [rows, ca + c])
                v10 = plsc.load_gather(rb_a1, [rows, cc])
                v11 = plsc.load_gather(rb_a1, [rows, ca + c])
                ta = v00 + fxa * (v01 - v00)
                ba = v10 + fxa * (v11 - v10)
                va = ta + fya * (ba - ta)
                u00 = plsc.load_gather(rb_b0, [rows, cc])
                u01 = plsc.load_gather(rb_b0, [rows, cb2 + c])
                u10 = plsc.load_gather(rb_b1, [rows, cc])
                u11 = plsc.load_gather(rb_b1, [rows, cb2 + c])
                tb = u00 + fxb * (u01 - u00)
                bb = u10 + fxb * (u11 - u10)
                vb = tb + fyb * (bb - tb)
                plsc.store_scatter(outb, [rows, cc], va + wl * (vb - va))
            return 0

        lax.fori_loop(0, C // 16, combine, 0)
        pltpu.sync_copy(outb, out_hbm.at[pl.ds(cb, C)])
        return 0

    lax.fori_loop(0, G, chunk, 0)


@jax.jit
def kernel(xs, scale, mip0, mip1, mip2, mip3, mip4, mip5, mip6, mip7, mip8,
           mip9, mip10):
    mips = (mip0, mip1, mip2, mip3, mip4, mip5, mip6, mip7, mip8, mip9, mip10)
    x = xs[:, 0]
    y = xs[:, 1]
    scale16 = jnp.full((16,), scale[0], dtype=jnp.float32)

    flats = [jnp.pad(m.reshape(-1, 3), ((0, 0), (0, 1))) for m in mips]
    flat = jnp.concatenate(flats, axis=0)
    ext = jnp.pad(flat, ((0, 3), (0, 0)))
    wtab = jnp.concatenate([ext[i:i + T] for i in range(4)], axis=1)

    mesh = plsc.VectorSubcoreMesh(core_axis_name="c", subcore_axis_name="s")
    fn = functools.partial(
        pl.kernel,
        mesh=mesh,
        compiler_params=pltpu.CompilerParams(needs_layout_passes=False, use_tc_tiling_on_sc=False),
        out_type=jax.ShapeDtypeStruct((N, 3), jnp.float32),
        scratch_types=[
            pltpu.VMEM((16,), jnp.float32),
            pltpu.VMEM((C,), jnp.float32),
            pltpu.VMEM((C,), jnp.float32),
            pltpu.VMEM((C,), jnp.int32),
            pltpu.VMEM((C,), jnp.int32),
            pltpu.VMEM((C,), jnp.int32),
            pltpu.VMEM((C,), jnp.int32),
            pltpu.VMEM((C, 16), jnp.float32),
            pltpu.VMEM((C, 16), jnp.float32),
            pltpu.VMEM((C, 16), jnp.float32),
            pltpu.VMEM((C, 16), jnp.float32),
            pltpu.VMEM((C,), jnp.float32),
            pltpu.VMEM((C,), jnp.float32),
            pltpu.VMEM((C,), jnp.float32),
            pltpu.VMEM((C,), jnp.float32),
            pltpu.VMEM((C,), jnp.int32),
            pltpu.VMEM((C,), jnp.int32),
            pltpu.VMEM((C, 3), jnp.float32),
            pltpu.SemaphoreType.DMA,
        ],
    )(_body)
    return fn(scale16, x, y, wtab)
